# baseline (speedup 1.0000x reference)
import functools

import jax
import jax.numpy as jnp
from jax import lax
from jax.experimental import pallas as pl
from jax.experimental.pallas import tpu as pltpu
from jax.experimental.pallas import tpu_sc as plsc

N_VERTS = 6890
N_FACES = 13776
B = 64

NC = 2
NS = 16
NW = NC * NS
L = 16
L2 = 2 * L

K = 40
IT0 = 16
IT1 = 6
NBUF = 4
DEPTH = 3
MAXIT = max(IT0, IT1)
ROWD = 3 * B // 2
NB = B // L
NG = 2


def _face_term(bp, bg, slot, k, g):
    o = g * L

    def ldrow(buf, r):
        return [plsc.bitcast(
            buf[slot, r, pl.ds(d * 2 * L + o, L)], jnp.bfloat16)
            for d in range(3)]

    p1 = ldrow(bp, k)
    p2 = ldrow(bp, K + k)
    p3 = ldrow(bp, 2 * K + k)
    g1 = ldrow(bg, k)
    g2 = ldrow(bg, K + k)
    g3 = ldrow(bg, 2 * K + k)
    e12p = (jnp.abs(p1[0] - p2[0]) + jnp.abs(p1[1] - p2[1])
            + jnp.abs(p1[2] - p2[2]))
    e13p = (jnp.abs(p1[0] - p3[0]) + jnp.abs(p1[1] - p3[1])
            + jnp.abs(p1[2] - p3[2]))
    e23p = (jnp.abs(p2[0] - p3[0]) + jnp.abs(p2[1] - p3[1])
            + jnp.abs(p2[2] - p3[2]))
    e12g = (jnp.abs(g1[0] - g2[0]) + jnp.abs(g1[1] - g2[1])
            + jnp.abs(g1[2] - g2[2]))
    e13g = (jnp.abs(g1[0] - g3[0]) + jnp.abs(g1[1] - g3[1])
            + jnp.abs(g1[2] - g3[2]))
    e23g = (jnp.abs(g2[0] - g3[0]) + jnp.abs(g2[1] - g3[1])
            + jnp.abs(g2[2] - g3[2]))
    return (jnp.abs(e12p - e12g) + jnp.abs(e13p - e13g)
            + jnp.abs(e23p - e23g))


def _edge_body(predt_hbm, gtt_hbm, idxs_hbm, mask_hbm, out_hbm,
               idx_v, bp_v, bg_v, mask_v, acc_v, out_v, *sems):
    cid = lax.axis_index("c")
    sid = lax.axis_index("s")
    w = sid * NC + cid

    pltpu.sync_copy(idxs_hbm.at[w], idx_v)
    pltpu.sync_copy(mask_hbm, mask_v)

    def start(it):
        slot = it % NBUF
        return (
            pltpu.async_copy(predt_hbm.at[idx_v.at[it]], bp_v.at[slot],
                             sems[slot]),
            pltpu.async_copy(gtt_hbm.at[idx_v.at[it]], bg_v.at[slot],
                             sems[slot]),
        )

    def run_chunks(iters):
        accs = tuple(jnp.zeros((L,), jnp.float32) for _ in range(NB))
        pend = {}
        for j in range(min(DEPTH, iters)):
            pend[j] = start(j)
        for it in range(iters):
            slot = it % NBUF
            cur = pend.pop(it)
            if it + DEPTH < iters:
                pend[it + DEPTH] = start(it + DEPTH)
            cur[0].wait()
            cur[1].wait()

            def face_body(k, accs, slot=slot):
                out = list(accs)
                for g in range(NG):
                    t = _face_term(bp_v, bg_v, slot, k, g)
                    ta, tb = plsc.unpack(
                        t, format=plsc.PackFormat.INTERLEAVED)
                    out[g * 2] = out[g * 2] + ta
                    out[g * 2 + 1] = out[g * 2 + 1] + tb
                return tuple(out)

            accs = lax.fori_loop(0, K, face_body, accs)
        for cc in range(NB):
            acc_v[cc, :] = accs[cc]

    @pl.when(cid == 0)
    def _():
        run_chunks(IT0)

    @pl.when(cid != 0)
    def _():
        run_chunks(IT1)

    part = acc_v[0, :] * mask_v[pl.ds(0, L)]
    msum = mask_v[pl.ds(0, L)]
    for cc in range(1, NB):
        part = part + acc_v[cc, :] * mask_v[pl.ds(cc * L, L)]
        msum = msum + mask_v[pl.ds(cc * L, L)]
    cs = jnp.flip(plsc.cumsum(msum))
    lane = lax.iota(jnp.int32, L)
    total = plsc.cumsum(jnp.where(lane == 0, cs, jnp.float32(0.0)))
    denom = total * jnp.float32(N_FACES)
    out_v[...] = part / denom
    pltpu.sync_copy(out_v, out_hbm.at[w])


@jax.jit
def _edge_loss(predt, gtt, idxs, maskf):
    mesh = plsc.VectorSubcoreMesh(core_axis_name="c", subcore_axis_name="s")
    run = functools.partial(
        pl.kernel,
        out_type=jax.ShapeDtypeStruct((NW, L), jnp.float32),
        mesh=mesh,
        compiler_params=pltpu.CompilerParams(
            needs_layout_passes=False, use_tc_tiling_on_sc=False),
        scratch_types=[
            pltpu.VMEM((MAXIT, 3 * K), jnp.int32),
            pltpu.VMEM((NBUF, 3 * K, ROWD), jnp.float32),
            pltpu.VMEM((NBUF, 3 * K, ROWD), jnp.float32),
            pltpu.VMEM((B,), jnp.float32),
            pltpu.VMEM((NB, L), jnp.float32),
            pltpu.VMEM((L,), jnp.float32),
        ] + [pltpu.SemaphoreType.DMA] * NBUF,
    )(_edge_body)
    out = run(predt, gtt, idxs, maskf)
    return jnp.sum(out)


def _pack(x):
    xh = x.astype(jnp.bfloat16)
    u = lax.bitcast_convert_type(xh, jnp.uint16).astype(jnp.uint32)
    words = u[:B // 2] | (u[B // 2:] << 16)
    return (lax.bitcast_convert_type(words, jnp.float32)
            .transpose(1, 2, 0).reshape(N_VERTS, ROWD))


def kernel(pred_verts, gt_verts, flag, faces):
    predt = _pack(pred_verts)
    gtt = _pack(gt_verts)
    f = faces.astype(jnp.int32)
    pad = NS * (IT0 + IT1) * K - N_FACES
    fp = jnp.concatenate([f, jnp.zeros((pad, 3), jnp.int32)], axis=0)
    n0 = NS * IT0 * K
    f0 = fp[:n0].reshape(NS, IT0, K, 3)
    f1 = fp[n0:].reshape(NS, IT1, K, 3)
    f1 = jnp.pad(f1, ((0, 0), (0, MAXIT - IT1), (0, 0), (0, 0)))
    f0 = jnp.pad(f0, ((0, 0), (0, MAXIT - IT0), (0, 0), (0, 0)))
    idxs = (jnp.stack([f0, f1], axis=1)
            .reshape(NW, MAXIT, K, 3)
            .transpose(0, 1, 3, 2)
            .reshape(NW, MAXIT, 3 * K))
    maskf = (flag == 1).astype(jnp.float32)
    maskp = maskf.reshape(2, NG, L).transpose(1, 0, 2).reshape(B)
    return _edge_loss(predt, gtt, idxs, maskp)


# speedup vs baseline: 1.2238x; 1.2238x over previous
import functools

import jax
import jax.numpy as jnp
from jax import lax
from jax.experimental import pallas as pl
from jax.experimental.pallas import tpu as pltpu
from jax.experimental.pallas import tpu_sc as plsc

N_VERTS = 6890
N_FACES = 13776
B = 64

NC = 2
NS = 16
NW = NC * NS
L = 16

K = 40
ITERS = 22
NBUF = 4
DEPTH = 3
ROWD = 3 * L * 2
NB = B // L
NROW_T = 432
NROW_TAIL = N_VERTS - (NS - 1) * NROW_T


def _face_term(buf, slot, k):
    def ldrow(r):
        return [plsc.bitcast(
            buf[slot, r, pl.ds(seg * L, L)], jnp.bfloat16)
            for seg in range(6)]

    v1 = ldrow(k)
    v2 = ldrow(K + k)
    v3 = ldrow(2 * K + k)
    e12p = (jnp.abs(v1[0] - v2[0]) + jnp.abs(v1[1] - v2[1])
            + jnp.abs(v1[2] - v2[2]))
    e13p = (jnp.abs(v1[0] - v3[0]) + jnp.abs(v1[1] - v3[1])
            + jnp.abs(v1[2] - v3[2]))
    e23p = (jnp.abs(v2[0] - v3[0]) + jnp.abs(v2[1] - v3[1])
            + jnp.abs(v2[2] - v3[2]))
    e12g = (jnp.abs(v1[3] - v2[3]) + jnp.abs(v1[4] - v2[4])
            + jnp.abs(v1[5] - v2[5]))
    e13g = (jnp.abs(v1[3] - v3[3]) + jnp.abs(v1[4] - v3[4])
            + jnp.abs(v1[5] - v3[5]))
    e23g = (jnp.abs(v2[3] - v3[3]) + jnp.abs(v2[4] - v3[4])
            + jnp.abs(v2[5] - v3[5]))
    return (jnp.abs(e12p - e12g) + jnp.abs(e13p - e13g)
            + jnp.abs(e23p - e23g))


def _edge_body(t0_hbm, t1_hbm, idxs_hbm, mask_hbm, out_hbm,
               idx_v, buf_v, mask_v, out_v, sh, *sems):
    cid = lax.axis_index("c")
    sid = lax.axis_index("s")
    w = sid * NC + cid

    def stage(src, n):
        base = sid * NROW_T
        return pltpu.async_copy(src.at[pl.ds(base, n)],
                                sh.at[pl.ds(base, n)], sems[0])

    for c, src in ((0, t0_hbm), (1, t1_hbm)):
        @pl.when((cid == c) & (sid < NS - 1))
        def _(src=src):
            stage(src, NROW_T).wait()

        @pl.when((cid == c) & (sid == NS - 1))
        def _(src=src):
            stage(src, NROW_TAIL).wait()

    pltpu.sync_copy(idxs_hbm.at[sid], idx_v)
    pltpu.sync_copy(mask_hbm, mask_v)
    plsc.subcore_barrier()

    def start(it):
        slot = it % NBUF
        return pltpu.async_copy(sh.at[idx_v.at[it]], buf_v.at[slot],
                                sems[slot])

    accs = (jnp.zeros((L,), jnp.float32), jnp.zeros((L,), jnp.float32))
    pend = {}
    for j in range(DEPTH):
        pend[j] = start(j)
    for it in range(ITERS):
        slot = it % NBUF
        cur = pend.pop(it)
        if it + DEPTH < ITERS:
            pend[it + DEPTH] = start(it + DEPTH)
        cur.wait()

        def face_body(k, accs, slot=slot):
            t = _face_term(buf_v, slot, k)
            ta, tb = plsc.unpack(t, format=plsc.PackFormat.INTERLEAVED)
            return (accs[0] + ta, accs[1] + tb)

        accs = lax.fori_loop(0, K, face_body, accs)

    half = cid * (2 * L)
    part0 = accs[0] * mask_v[pl.ds(half, L)]
    part1 = accs[1] * mask_v[pl.ds(half + L, L)]
    msum = mask_v[pl.ds(0, L)]
    for cc in range(1, NB):
        msum = msum + mask_v[pl.ds(cc * L, L)]
    cs = jnp.flip(plsc.cumsum(msum))
    lane = lax.iota(jnp.int32, L)
    total = plsc.cumsum(jnp.where(lane == 0, cs, jnp.float32(0.0)))
    denom = total * jnp.float32(N_FACES)
    out_v[0, :] = part0 / denom
    out_v[1, :] = part1 / denom
    pltpu.sync_copy(out_v, out_hbm.at[w])


@jax.jit
def _edge_loss(t0, t1, idxs, maskf):
    mesh = plsc.VectorSubcoreMesh(core_axis_name="c", subcore_axis_name="s")
    run = functools.partial(
        pl.kernel,
        out_type=jax.ShapeDtypeStruct((NW, 2, L), jnp.float32),
        mesh=mesh,
        compiler_params=pltpu.CompilerParams(
            needs_layout_passes=False, use_tc_tiling_on_sc=False),
        scratch_types=[
            pltpu.VMEM((ITERS, 3 * K), jnp.int32),
            pltpu.VMEM((NBUF, 3 * K, ROWD), jnp.float32),
            pltpu.VMEM((B,), jnp.float32),
            pltpu.VMEM((2, L), jnp.float32),
            pltpu.VMEM_SHARED((N_VERTS, ROWD), jnp.float32),
        ] + [pltpu.SemaphoreType.DMA] * NBUF,
    )(_edge_body)
    out = run(t0, t1, idxs, maskf)
    return jnp.sum(out)


def _packh(x, h):
    xh = x.astype(jnp.bfloat16)
    u = lax.bitcast_convert_type(xh, jnp.uint16).astype(jnp.uint32)
    lo = u[h * 2 * L:h * 2 * L + L]
    hi = u[h * 2 * L + L:(h + 1) * 2 * L]
    words = lo | (hi << 16)
    return (lax.bitcast_convert_type(words, jnp.float32)
            .transpose(1, 2, 0).reshape(N_VERTS, 3 * L))


def kernel(pred_verts, gt_verts, flag, faces):
    t0 = jnp.concatenate([_packh(pred_verts, 0), _packh(gt_verts, 0)],
                         axis=1)
    t1 = jnp.concatenate([_packh(pred_verts, 1), _packh(gt_verts, 1)],
                         axis=1)
    f = faces.astype(jnp.int32)
    pad = NS * ITERS * K - N_FACES
    fp = jnp.concatenate([f, jnp.zeros((pad, 3), jnp.int32)], axis=0)
    idxs = (fp.reshape(NS, ITERS, K, 3)
            .transpose(0, 1, 3, 2)
            .reshape(NS, ITERS, 3 * K))
    maskf = (flag == 1).astype(jnp.float32)
    return _edge_loss(t0, t1, idxs, maskf)


# speedup vs baseline: 1.3678x; 1.1177x over previous
import functools

import jax
import jax.numpy as jnp
from jax import lax
from jax.experimental import pallas as pl
from jax.experimental.pallas import tpu as pltpu
from jax.experimental.pallas import tpu_sc as plsc

N_VERTS = 6890
N_FACES = 13776
B = 64

NC = 2
NS = 16
NW = NC * NS
L = 16

K = 40
ITERS = 22
NBUF = 4
DEPTH = 3
ROWD = 3 * L
NB = B // L
NROW_T = 432
NROW_TAIL = N_VERTS - (NS - 1) * NROW_T


def _face_term(bp, bg, slot, k):
    def ldrow(buf, r):
        return [plsc.bitcast(
            buf[slot, r, pl.ds(d * L, L)], jnp.bfloat16)
            for d in range(3)]

    v1 = ldrow(bp, k) + ldrow(bg, k)
    v2 = ldrow(bp, K + k) + ldrow(bg, K + k)
    v3 = ldrow(bp, 2 * K + k) + ldrow(bg, 2 * K + k)
    e12p = (jnp.abs(v1[0] - v2[0]) + jnp.abs(v1[1] - v2[1])
            + jnp.abs(v1[2] - v2[2]))
    e13p = (jnp.abs(v1[0] - v3[0]) + jnp.abs(v1[1] - v3[1])
            + jnp.abs(v1[2] - v3[2]))
    e23p = (jnp.abs(v2[0] - v3[0]) + jnp.abs(v2[1] - v3[1])
            + jnp.abs(v2[2] - v3[2]))
    e12g = (jnp.abs(v1[3] - v2[3]) + jnp.abs(v1[4] - v2[4])
            + jnp.abs(v1[5] - v2[5]))
    e13g = (jnp.abs(v1[3] - v3[3]) + jnp.abs(v1[4] - v3[4])
            + jnp.abs(v1[5] - v3[5]))
    e23g = (jnp.abs(v2[3] - v3[3]) + jnp.abs(v2[4] - v3[4])
            + jnp.abs(v2[5] - v3[5]))
    return (jnp.abs(e12p - e12g) + jnp.abs(e13p - e13g)
            + jnp.abs(e23p - e23g))


def _edge_body(p0_hbm, g0_hbm, p1_hbm, g1_hbm, idxs_hbm, mask_hbm, out_hbm,
               idx_v, bp_v, bg_v, mask_v, out_v, shp, shg, *sems):
    cid = lax.axis_index("c")
    sid = lax.axis_index("s")
    w = sid * NC + cid

    def stage(srcp, srcg, n):
        base = sid * NROW_T
        c1 = pltpu.async_copy(srcp.at[pl.ds(base, n)],
                              shp.at[pl.ds(base, n)], sems[0])
        c2 = pltpu.async_copy(srcg.at[pl.ds(base, n)],
                              shg.at[pl.ds(base, n)], sems[1])
        c1.wait()
        c2.wait()

    for c, srcp, srcg in ((0, p0_hbm, g0_hbm), (1, p1_hbm, g1_hbm)):
        @pl.when((cid == c) & (sid < NS - 1))
        def _(srcp=srcp, srcg=srcg):
            stage(srcp, srcg, NROW_T)

        @pl.when((cid == c) & (sid == NS - 1))
        def _(srcp=srcp, srcg=srcg):
            stage(srcp, srcg, NROW_TAIL)

    pltpu.sync_copy(idxs_hbm.at[sid], idx_v)
    pltpu.sync_copy(mask_hbm, mask_v)
    plsc.subcore_barrier()

    def start(it):
        slot = it % NBUF
        return (
            pltpu.async_copy(shp.at[idx_v.at[it]], bp_v.at[slot],
                             sems[slot]),
            pltpu.async_copy(shg.at[idx_v.at[it]], bg_v.at[slot],
                             sems[slot]),
        )

    accs = (jnp.zeros((L,), jnp.float32), jnp.zeros((L,), jnp.float32))
    pend = {}
    for j in range(DEPTH):
        pend[j] = start(j)
    for it in range(ITERS):
        slot = it % NBUF
        cur = pend.pop(it)
        if it + DEPTH < ITERS:
            pend[it + DEPTH] = start(it + DEPTH)
        cur[0].wait()
        cur[1].wait()

        def face_body(k, accs, slot=slot):
            t = _face_term(bp_v, bg_v, slot, k)
            ta, tb = plsc.unpack(t, format=plsc.PackFormat.INTERLEAVED)
            return (accs[0] + ta, accs[1] + tb)

        accs = lax.fori_loop(0, K, face_body, accs)

    half = cid * (2 * L)
    part0 = accs[0] * mask_v[pl.ds(half, L)]
    part1 = accs[1] * mask_v[pl.ds(half + L, L)]
    msum = mask_v[pl.ds(0, L)]
    for cc in range(1, NB):
        msum = msum + mask_v[pl.ds(cc * L, L)]
    cs = jnp.flip(plsc.cumsum(msum))
    lane = lax.iota(jnp.int32, L)
    total = plsc.cumsum(jnp.where(lane == 0, cs, jnp.float32(0.0)))
    denom = total * jnp.float32(N_FACES)
    out_v[0, :] = part0 / denom
    out_v[1, :] = part1 / denom
    pltpu.sync_copy(out_v, out_hbm.at[w])


@jax.jit
def _edge_loss(p0, g0, p1, g1, idxs, maskf):
    mesh = plsc.VectorSubcoreMesh(core_axis_name="c", subcore_axis_name="s")
    run = functools.partial(
        pl.kernel,
        out_type=jax.ShapeDtypeStruct((NW, 2, L), jnp.float32),
        mesh=mesh,
        compiler_params=pltpu.CompilerParams(
            needs_layout_passes=False, use_tc_tiling_on_sc=False),
        scratch_types=[
            pltpu.VMEM((ITERS, 3 * K), jnp.int32),
            pltpu.VMEM((NBUF, 3 * K, ROWD), jnp.float32),
            pltpu.VMEM((NBUF, 3 * K, ROWD), jnp.float32),
            pltpu.VMEM((B,), jnp.float32),
            pltpu.VMEM((2, L), jnp.float32),
            pltpu.VMEM_SHARED((N_VERTS, ROWD), jnp.float32),
            pltpu.VMEM_SHARED((N_VERTS, ROWD), jnp.float32),
        ] + [pltpu.SemaphoreType.DMA] * NBUF,
    )(_edge_body)
    out = run(p0, g0, p1, g1, idxs, maskf)
    return jnp.sum(out)


def _packh(x, h):
    xh = x.astype(jnp.bfloat16)
    u = lax.bitcast_convert_type(xh, jnp.uint16).astype(jnp.uint32)
    lo = u[h * 2 * L:h * 2 * L + L]
    hi = u[h * 2 * L + L:(h + 1) * 2 * L]
    words = lo | (hi << 16)
    return (lax.bitcast_convert_type(words, jnp.float32)
            .transpose(1, 2, 0).reshape(N_VERTS, 3 * L))


def kernel(pred_verts, gt_verts, flag, faces):
    p0 = _packh(pred_verts, 0)
    g0 = _packh(gt_verts, 0)
    p1 = _packh(pred_verts, 1)
    g1 = _packh(gt_verts, 1)
    f = faces.astype(jnp.int32)
    pad = NS * ITERS * K - N_FACES
    fp = jnp.concatenate([f, jnp.zeros((pad, 3), jnp.int32)], axis=0)
    idxs = (fp.reshape(NS, ITERS, K, 3)
            .transpose(0, 1, 3, 2)
            .reshape(NS, ITERS, 3 * K))
    maskf = (flag == 1).astype(jnp.float32)
    return _edge_loss(p0, g0, p1, g1, idxs, maskf)
